# Initial kernel scaffold; baseline (speedup 1.0000x reference)
import jax
import jax.numpy as jnp
from jax.experimental import pallas as pl


def kernel(hidden_states, gate_weight, gate_bias, gate_up_proj, gate_up_proj_bias, down_proj, down_proj_bias):
    raise NotImplementedError("write your pallas kernel here")


# speedup vs baseline: 2.5857x; 2.5857x over previous
import functools

import jax
import jax.numpy as jnp
from jax import lax
from jax.experimental import pallas as pl
from jax.experimental.pallas import tpu as pltpu
from jax.experimental.pallas import tpu_sc as plsc

S = 2048
H = 1024
E = 16
FF = 2880
NPAIR = 2 * S
BLK = 256
NBLK = NPAIR // BLK + E
NSLOT = NBLK * BLK

NW = 32


def _router_body(x_ref, gw_ref, gb_ref,
                 logits_ref, pos_ref, ww_ref, meta_ref,
                 ind_ref, cum_ref):
    x = x_ref[...]
    gw = gw_ref[...]
    logits = lax.dot_general(x, gw, (((1,), (1,)), ((), ())),
                             preferred_element_type=jnp.float32) + gb_ref[...]
    logits_ref[...] = logits

    eio = lax.broadcasted_iota(jnp.int32, (S, E), 1)
    v1 = jnp.max(logits, axis=1, keepdims=True)
    a1 = jnp.min(jnp.where(logits == v1, eio, E), axis=1, keepdims=True)
    masked = jnp.where(eio == a1, -jnp.inf, logits)
    v2 = jnp.max(masked, axis=1, keepdims=True)
    a2 = jnp.min(jnp.where(masked == v2, eio, E), axis=1, keepdims=True)

    t = jnp.exp(v2 - v1)
    w1 = 1.0 / (1.0 + t)
    ww_ref[0:S, :] = jnp.broadcast_to(w1, (S, E))
    ww_ref[S:NPAIR, :] = jnp.broadcast_to(1.0 - w1, (S, E))

    ind_ref[0:S, :] = (eio == a1).astype(jnp.float32)
    ind_ref[S:NPAIR, :] = (eio == a2).astype(jnp.float32)

    tri = (lax.broadcasted_iota(jnp.int32, (128, 128), 0)
           >= lax.broadcasted_iota(jnp.int32, (128, 128), 1)).astype(jnp.float32)

    def chunk(c, off):
        blk = ind_ref[pl.ds(c * 128, 128), :]
        cum = lax.dot_general(tri, blk, (((1,), (0,)), ((), ())),
                              preferred_element_type=jnp.float32) + off
        cum_ref[pl.ds(c * 128, 128), :] = cum
        return cum[127:128, :]

    counts = lax.fori_loop(0, NPAIR // 128, chunk, jnp.zeros((1, E), jnp.float32))

    counts_i = counts.astype(jnp.int32)
    nb = (counts_i + (BLK - 1)) // BLK
    padded = (nb * BLK).astype(jnp.float32)
    upper = (lax.broadcasted_iota(jnp.int32, (E, E), 0)
             < lax.broadcasted_iota(jnp.int32, (E, E), 1)).astype(jnp.float32)
    bs = lax.dot_general(padded, upper, (((1,), (0,)), ((), ())),
                         preferred_element_type=jnp.float32)

    pos = jnp.sum((cum_ref[...] + bs) * ind_ref[...], axis=1, keepdims=True) - 1.0
    pos_ref[...] = pos.astype(jnp.int32)

    bs_blk = bs.astype(jnp.int32) // BLK
    bio = lax.broadcasted_iota(jnp.int32, (NBLK, E), 0)
    er = lax.broadcasted_iota(jnp.int32, (NBLK, E), 1)
    act = jnp.logical_and(bio >= bs_blk, bio < bs_blk + nb)
    be = jnp.sum(jnp.where(act, er, 0), axis=1, keepdims=True)
    valid = jnp.sum(jnp.where(act, 1, 0), axis=1, keepdims=True)
    er16 = lax.broadcasted_iota(jnp.int32, (1, E), 1)
    e_last = jnp.max(jnp.where(nb > 0, er16, 0))
    meta_ref[:, 0:1] = jnp.where(valid > 0, be, e_last)
    meta_ref[:, 1:2] = valid


def _router(x, gw, gb):
    return pl.pallas_call(
        _router_body,
        out_shape=(
            jax.ShapeDtypeStruct((S, E), jnp.float32),
            jax.ShapeDtypeStruct((NPAIR, 1), jnp.int32),
            jax.ShapeDtypeStruct((NPAIR, E), jnp.float32),
            jax.ShapeDtypeStruct((NBLK, 2), jnp.int32),
        ),
        scratch_shapes=[
            pltpu.VMEM((NPAIR, E), jnp.float32),
            pltpu.VMEM((NPAIR, E), jnp.float32),
        ],
    )(x, gw, gb)


def _dispatch_body(x_hbm, pos_hbm, xg_hbm, idx_v, rows_v, sem):
    wid = lax.axis_index("s") * 2 + lax.axis_index("c")
    for c in range(2):
        p0 = pl.multiple_of(wid * 128 + c * 64, 64)
        t0 = pl.multiple_of(lax.rem(p0, S), 64)
        pltpu.sync_copy(pos_hbm.at[pl.ds(p0, 64)], idx_v)
        pltpu.sync_copy(x_hbm.at[pl.ds(t0, 64)], rows_v)
        pltpu.async_copy(rows_v, xg_hbm.at[idx_v], sem).wait()


def _dispatch(x, pos):
    mesh = plsc.VectorSubcoreMesh(core_axis_name="c", subcore_axis_name="s")
    return pl.kernel(
        _dispatch_body,
        out_type=jax.ShapeDtypeStruct((NSLOT, H), jnp.float32),
        mesh=mesh,
        scratch_types=[
            pltpu.VMEM((64,), jnp.int32),
            pltpu.VMEM((64, H), jnp.float32),
            pltpu.SemaphoreType.DMA,
        ],
    )(x, pos)


GUT = 1920
NGU = 2 * FF // GUT


def _ffn_body(meta_sref, xg_ref, gu_ref, gub_ref, dp_ref, db_ref, ys_ref, acc_ref):
    b = pl.program_id(0)
    j = pl.program_id(1)

    @pl.when(meta_sref[b, 1] > 0)
    def _():
        x = xg_ref[...]
        part = jnp.dot(x, gu_ref[0], preferred_element_type=jnp.float32)
        for jj in range(NGU):
            @pl.when(j == jj)
            def _():
                acc_ref[:, jj * GUT:(jj + 1) * GUT] = part

        @pl.when(j == NGU - 1)
        def _():
            gu = acc_ref[...] + gub_ref[0]
            g = gu[:, :FF]
            u = gu[:, FF:]
            inter = g * jax.nn.sigmoid(g) * u
            ys_ref[...] = (jnp.dot(inter, dp_ref[0],
                                   preferred_element_type=jnp.float32) + db_ref[0])


def _ffn(meta, xg, gup, gub, dp, db):
    grid_spec = pltpu.PrefetchScalarGridSpec(
        num_scalar_prefetch=1,
        grid=(NBLK, NGU),
        in_specs=[
            pl.BlockSpec((BLK, H), lambda b, j, m: (b, 0)),
            pl.BlockSpec((1, H, GUT), lambda b, j, m: (m[b, 0], 0, j)),
            pl.BlockSpec((1, 1, 2 * FF), lambda b, j, m: (m[b, 0], 0, 0)),
            pl.BlockSpec((1, FF, H), lambda b, j, m: (m[b, 0], 0, 0)),
            pl.BlockSpec((1, 1, H), lambda b, j, m: (m[b, 0], 0, 0)),
        ],
        out_specs=pl.BlockSpec((BLK, H), lambda b, j, m: (b, 0)),
        scratch_shapes=[pltpu.VMEM((BLK, 2 * FF), jnp.float32)],
    )
    return pl.pallas_call(
        _ffn_body,
        grid_spec=grid_spec,
        out_shape=jax.ShapeDtypeStruct((NSLOT, H), jnp.float32),
        compiler_params=pltpu.CompilerParams(
            dimension_semantics=("arbitrary", "arbitrary"),
        ),
    )(meta, xg, gup, gub.reshape(E, 1, 2 * FF), dp, db.reshape(E, 1, H))


def _combine_body(ys_hbm, pos_hbm, ww_hbm, out_hbm,
                  idx0_v, idx1_v, r0_v, r1_v, w0_v, w1_v, sem0, sem1):
    wid = lax.axis_index("s") * 2 + lax.axis_index("c")
    for c in range(2):
        base = pl.multiple_of(wid * 64 + c * 32, 32)
        pltpu.sync_copy(pos_hbm.at[pl.ds(base, 32)], idx0_v)
        pltpu.sync_copy(pos_hbm.at[pl.ds(base + S, 32)], idx1_v)
        pltpu.sync_copy(ww_hbm.at[pl.ds(base, 32)], w0_v)
        pltpu.sync_copy(ww_hbm.at[pl.ds(base + S, 32)], w1_v)
        cp0 = pltpu.async_copy(ys_hbm.at[idx0_v], r0_v, sem0)
        cp1 = pltpu.async_copy(ys_hbm.at[idx1_v], r1_v, sem1)
        cp0.wait()
        cp1.wait()
        for i in range(32):
            w0 = w0_v[i, :]
            w1 = w1_v[i, :]

            def lane(j, _):
                sl = pl.ds(j * 16, 16)
                r0_v[i, sl] = w0 * r0_v[i, sl] + w1 * r1_v[i, sl]
                return 0

            lax.fori_loop(0, H // 16, lane, 0)
        pltpu.sync_copy(r0_v, out_hbm.at[pl.ds(base, 32)])


def _combine(ys, pos, ww):
    mesh = plsc.VectorSubcoreMesh(core_axis_name="c", subcore_axis_name="s")
    return pl.kernel(
        _combine_body,
        out_type=jax.ShapeDtypeStruct((S, H), jnp.float32),
        mesh=mesh,
        scratch_types=[
            pltpu.VMEM((32,), jnp.int32),
            pltpu.VMEM((32,), jnp.int32),
            pltpu.VMEM((32, H), jnp.float32),
            pltpu.VMEM((32, H), jnp.float32),
            pltpu.VMEM((32, 16), jnp.float32),
            pltpu.VMEM((32, 16), jnp.float32),
            pltpu.SemaphoreType.DMA,
            pltpu.SemaphoreType.DMA,
        ],
    )(ys, pos, ww)


def kernel(hidden_states, gate_weight, gate_bias, gate_up_proj,
           gate_up_proj_bias, down_proj, down_proj_bias):
    b, s, hd = hidden_states.shape
    x = hidden_states.reshape(s, hd)

    logits, pos, ww, meta = _router(x, gate_weight, gate_bias.reshape(1, E))
    xg = _dispatch(x, pos.reshape(NPAIR))
    ys = _ffn(meta, xg, gate_up_proj, gate_up_proj_bias, down_proj, down_proj_bias)
    out = _combine(ys, pos.reshape(NPAIR), ww)
    return out.reshape(b, s, hd), logits
